# Initial kernel scaffold; baseline (speedup 1.0000x reference)
import jax
import jax.numpy as jnp
from jax.experimental import pallas as pl


def kernel(pred, target):
    raise NotImplementedError("write your pallas kernel here")


# speedup vs baseline: 1.0022x; 1.0022x over previous
import jax
import jax.numpy as jnp
from jax.experimental import pallas as pl
from jax.experimental.pallas import tpu as pltpu

_N = 8192
_BLOCK_ROWS = 256
_GRID = _N // _BLOCK_ROWS
_SCALE = 1.0 / (_N * _N)


def _body(p_ref, t_ref, o_ref):
    i = pl.program_id(0)
    p = p_ref[...]
    t = t_ref[...]
    q = jnp.where(t == 0, p - 0.1, 1.0 - p)
    q = jnp.clip(q, 0.0, 1.0)
    s = jnp.sum(q * q)

    @pl.when(i == 0)
    def _init():
        o_ref[0, 0] = 0.0

    o_ref[0, 0] += s

    @pl.when(i == _GRID - 1)
    def _final():
        o_ref[0, 0] = o_ref[0, 0] * _SCALE


def kernel(pred, target):
    out = pl.pallas_call(
        _body,
        grid=(_GRID,),
        in_specs=[
            pl.BlockSpec((_BLOCK_ROWS, _N), lambda i: (i, 0)),
            pl.BlockSpec((_BLOCK_ROWS, _N), lambda i: (i, 0)),
        ],
        out_specs=pl.BlockSpec(memory_space=pltpu.SMEM),
        out_shape=jax.ShapeDtypeStruct((1, 1), jnp.float32),
        compiler_params=pltpu.CompilerParams(
            dimension_semantics=("arbitrary",),
        ),
    )(pred, target)
    return out[0, 0]
